# Initial kernel scaffold; baseline (speedup 1.0000x reference)
import jax
import jax.numpy as jnp
from jax.experimental import pallas as pl


def kernel(features, W1, bn_gamma, bn_beta, bn_mean, bn_var, W2, b2):
    raise NotImplementedError("write your pallas kernel here")


# speedup vs baseline: 1.0891x; 1.0891x over previous
import functools

import jax
import jax.numpy as jnp
from jax.experimental import pallas as pl

B, CIN, SY, SX = 8, 384, 32, 32
A, NC = 5, 80
HID = 1024
OUT_CH = A * (5 + NC)
P = SY * SX
PAD = 64
PL = P + 2 * PAD


def _head_kernel(x_ref, w1_ref, b1_ref, w2_ref, b2_ref, out_ref):
    x = x_ref[0]
    pos = jax.lax.broadcasted_iota(jnp.int32, (P, 1), 0)
    xcol = pos % SX
    mask_left = (xcol != 0)
    mask_right = (xcol != SX - 1)

    acc = jnp.zeros((P, HID), dtype=jnp.float32)
    for ky in range(3):
        for kx in range(3):
            t = ky * 3 + kx
            s = PAD + SX * (ky - 1) + (kx - 1)
            blk = x[s:s + P, :]
            if kx == 0:
                blk = jnp.where(mask_left, blk, jnp.bfloat16(0))
            elif kx == 2:
                blk = jnp.where(mask_right, blk, jnp.bfloat16(0))
            acc += jnp.dot(blk, w1_ref[t],
                           preferred_element_type=jnp.float32)

    h = acc + b1_ref[...]
    h = jnp.where(h > 0, h, 0.1 * h)
    out = jnp.dot(h.astype(jnp.bfloat16), w2_ref[...],
                  preferred_element_type=jnp.float32)
    out_ref[0] = out + b2_ref[...]


@functools.partial(jax.jit, static_argnums=())
def kernel(features, W1, bn_gamma, bn_beta, bn_mean, bn_var, W2, b2):
    eps = 1e-5
    scale = bn_gamma * jax.lax.rsqrt(bn_var + eps)
    bias1 = bn_beta - bn_mean * scale
    w1 = (W1 * scale[:, None, None, None])
    w1 = jnp.transpose(w1, (2, 3, 1, 0)).reshape(9, CIN, HID)
    w1 = w1.astype(jnp.bfloat16)
    w2 = W2[:, :, 0, 0].T.astype(jnp.bfloat16)

    xt = jnp.transpose(features, (0, 2, 3, 1)).reshape(B, P, CIN)
    xp = jnp.pad(xt, ((0, 0), (PAD, PAD), (0, 0))).astype(jnp.bfloat16)

    out = pl.pallas_call(
        _head_kernel,
        grid=(B,),
        in_specs=[
            pl.BlockSpec((1, PL, CIN), lambda b: (b, 0, 0)),
            pl.BlockSpec((9, CIN, HID), lambda b: (0, 0, 0)),
            pl.BlockSpec((1, HID), lambda b: (0, 0)),
            pl.BlockSpec((HID, OUT_CH), lambda b: (0, 0)),
            pl.BlockSpec((1, OUT_CH), lambda b: (0, 0)),
        ],
        out_specs=pl.BlockSpec((1, P, OUT_CH), lambda b: (b, 0, 0)),
        out_shape=jax.ShapeDtypeStruct((B, P, OUT_CH), jnp.float32),
    )(xp, w1, bias1[None, :], w2, b2[None, :])
    return out.reshape(B, SY, SX, OUT_CH)


# speedup vs baseline: 1.2712x; 1.1672x over previous
import jax
import jax.numpy as jnp
from jax.experimental import pallas as pl

B, CIN, SY, SX = 8, 384, 32, 32
A, NC = 5, 80
HID = 1024
OUT_CH = A * (5 + NC)
P = SY * SX
PAD = 64
PL = P + 2 * PAD


def _head_kernel(x_ref, w1_ref, s1_ref, b1_ref, w2_ref, b2_ref, out_ref):
    x = x_ref[0]
    pos = jax.lax.broadcasted_iota(jnp.int32, (P, 1), 0)
    xcol = pos % SX
    mask_left = (xcol != 0)
    mask_right = (xcol != SX - 1)

    cols = []
    for ky in range(3):
        for kx in range(3):
            s = PAD + SX * (ky - 1) + (kx - 1)
            blk = x[s:s + P, :]
            if kx == 0:
                blk = jnp.where(mask_left, blk, jnp.bfloat16(0))
            elif kx == 2:
                blk = jnp.where(mask_right, blk, jnp.bfloat16(0))
            cols.append(blk)
    im2 = jnp.concatenate(cols, axis=1)

    acc = jnp.dot(im2, w1_ref[...], preferred_element_type=jnp.float32)
    h = acc * s1_ref[...] + b1_ref[...]
    h = jnp.where(h > 0, h, 0.1 * h)
    out = jnp.dot(h.astype(jnp.bfloat16), w2_ref[...],
                  preferred_element_type=jnp.float32)
    out_ref[0] = out + b2_ref[...]


def kernel(features, W1, bn_gamma, bn_beta, bn_mean, bn_var, W2, b2):
    eps = 1e-5
    scale = bn_gamma * jax.lax.rsqrt(bn_var + eps)
    bias1 = bn_beta - bn_mean * scale
    w1 = jnp.transpose(W1.astype(jnp.bfloat16), (2, 3, 1, 0))
    w1 = w1.reshape(9 * CIN, HID)
    w2 = W2[:, :, 0, 0].T.astype(jnp.bfloat16)

    xt = jnp.transpose(features.astype(jnp.bfloat16), (0, 2, 3, 1))
    xp = jnp.pad(xt.reshape(B, P, CIN), ((0, 0), (PAD, PAD), (0, 0)))

    out = pl.pallas_call(
        _head_kernel,
        grid=(B,),
        in_specs=[
            pl.BlockSpec((1, PL, CIN), lambda b: (b, 0, 0)),
            pl.BlockSpec((9 * CIN, HID), lambda b: (0, 0)),
            pl.BlockSpec((1, HID), lambda b: (0, 0)),
            pl.BlockSpec((1, HID), lambda b: (0, 0)),
            pl.BlockSpec((HID, OUT_CH), lambda b: (0, 0)),
            pl.BlockSpec((1, OUT_CH), lambda b: (0, 0)),
        ],
        out_specs=pl.BlockSpec((1, P, OUT_CH), lambda b: (b, 0, 0)),
        out_shape=jax.ShapeDtypeStruct((B, P, OUT_CH), jnp.float32),
    )(xp, w1, scale[None, :], bias1[None, :], w2, b2[None, :])
    return out.reshape(B, SY, SX, OUT_CH)
